# baseline (speedup 1.0000x reference)
import functools

import jax
import jax.numpy as jnp
from jax import lax
from jax.experimental import pallas as pl
from jax.experimental.pallas import tpu as pltpu
from jax.experimental.pallas import tpu_sc as plsc

_B, _A = 16384, 512
_S = 3072
_NC, _NS, _L = 2, 16, 16
_NW = _NC * _NS
_ROWS_W = _S // _NW
_CR = 16
_NCHUNK = _ROWS_W // _CR
_NPAIR = _NCHUNK // 2
_NGRP = (_CR // 4) * (_A // _L)
_BR = 512


def _body(loc_hbm, tgt_hbm, msk_hbm, out_hbm,
          loc_v0, loc_v1, tgt_v0, tgt_v1, msk_v0, msk_v1,
          in_sem0, in_sem1, out_sem0, out_sem1):
    wid = lax.axis_index("s") * _NC + lax.axis_index("c")
    base = wid * _ROWS_W
    loc_v = (loc_v0, loc_v1)
    tgt_v = (tgt_v0, tgt_v1)
    msk_v = (msk_v0, msk_v1)
    in_sem = (in_sem0, in_sem1)
    out_sem = (out_sem0, out_sem1)

    def row_off(i):
        return pl.multiple_of(base + i * _CR, _CR)

    def in_copies(i, p):
        r0 = row_off(i)
        return (
            pltpu.make_async_copy(loc_hbm.at[pl.ds(r0, _CR), :], loc_v[p], in_sem[p]),
            pltpu.make_async_copy(tgt_hbm.at[pl.ds(r0, _CR), :], tgt_v[p], in_sem[p]),
            pltpu.make_async_copy(msk_hbm.at[pl.ds(r0, _CR), :], msk_v[p], in_sem[p]),
        )

    def start_in(i, p):
        for d in in_copies(i, p):
            d.start()

    def wait_in(i, p):
        for d in in_copies(i, p):
            d.wait()

    def out_copy(i, p):
        return pltpu.make_async_copy(
            loc_v[p], out_hbm.at[pl.ds(row_off(i), _CR), :], out_sem[p])

    def compute(p):
        lv, tv = loc_v[p], tgt_v[p]
        mw = msk_v[p].bitcast(jnp.int32)

        @plsc.parallel_loop(0, _NGRP, unroll=4)
        def grp_body(g):
            s = g >> 5
            c = (g & 31) * _L
            m_words = mw[s, pl.ds(c, _L)]
            for k in range(4):
                r = s * 4 + k
                sl = pl.ds(c, _L)
                m8 = lax.shift_right_logical(m_words, 8 * k) & 0xFF
                l = lv[r, sl]
                t = tv[r, sl]
                lv[r, sl] = jnp.where(m8 != 0, t, l)

    start_in(0, 0)

    def pair_body(j, carry):
        a = j * 2
        wait_in(a, 0)

        @pl.when(j > 0)
        def _():
            out_copy(a - 2, 0).wait()

        start_in(a + 1, 1)
        compute(0)
        out_copy(a, 0).start()

        wait_in(a + 1, 1)

        @pl.when(j > 0)
        def _():
            out_copy(a - 1, 1).wait()

        @pl.when(j < _NPAIR - 1)
        def _():
            start_in(a + 2, 0)

        compute(1)
        out_copy(a + 1, 1).start()
        return carry

    lax.fori_loop(0, _NPAIR, pair_body, 0)
    out_copy(_NCHUNK - 2, 0).wait()
    out_copy(_NCHUNK - 1, 1).wait()


def _tc_body(l_ref, t_ref, m_ref, o_ref):
    o_ref[...] = jnp.where(m_ref[...], t_ref[...], l_ref[...])


@jax.jit
def kernel(location, movement_targets, movement_mask):
    mesh = plsc.VectorSubcoreMesh(core_axis_name="c", subcore_axis_name="s")
    out_sc = pl.kernel(
        _body,
        mesh=mesh,
        out_type=jax.ShapeDtypeStruct((_S, _A), jnp.int32),
        scratch_types=[
            pltpu.VMEM((_CR, _A), jnp.int32),
            pltpu.VMEM((_CR, _A), jnp.int32),
            pltpu.VMEM((_CR, _A), jnp.int32),
            pltpu.VMEM((_CR, _A), jnp.int32),
            pltpu.VMEM((_CR, _A), jnp.uint8),
            pltpu.VMEM((_CR, _A), jnp.uint8),
            pltpu.SemaphoreType.DMA,
            pltpu.SemaphoreType.DMA,
            pltpu.SemaphoreType.DMA,
            pltpu.SemaphoreType.DMA,
        ],
        compiler_params=pltpu.CompilerParams(use_tc_tiling_on_sc=True),
    )(location, movement_targets, movement_mask.view(jnp.uint8))

    nblk = _S // _BR
    out_tc = pl.pallas_call(
        _tc_body,
        grid=((_B - _S) // _BR,),
        in_specs=[
            pl.BlockSpec((_BR, _A), lambda i: (nblk + i, 0)),
            pl.BlockSpec((_BR, _A), lambda i: (nblk + i, 0)),
            pl.BlockSpec((_BR, _A), lambda i: (nblk + i, 0)),
        ],
        out_specs=pl.BlockSpec((_BR, _A), lambda i: (nblk + i, 0)),
        out_shape=jax.ShapeDtypeStruct((_B, _A), jnp.int32),
    )(location, movement_targets, movement_mask)

    return lax.dynamic_update_slice(out_tc, out_sc, (0, 0))


# speedup vs baseline: 1.1959x; 1.1959x over previous
import functools

import jax
import jax.numpy as jnp
from jax import lax
from jax.experimental import pallas as pl
from jax.experimental.pallas import tpu as pltpu
from jax.experimental.pallas import tpu_sc as plsc

_B, _A = 16384, 512
_NC, _NS, _L = 2, 16, 16
_NW = _NC * _NS
_ROWS_W = _B // _NW
_CR = 32
_NCHUNK = _ROWS_W // _CR
_NPAIR = _NCHUNK // 2
_NGRP = (_CR // 4) * (_A // _L)


def _body(loc_hbm, tgt_hbm, msk_hbm, out_hbm,
          loc_v0, loc_v1, tgt_v0, tgt_v1, msk_v0, msk_v1,
          in_sem0, in_sem1, out_sem0, out_sem1):
    wid = lax.axis_index("s") * _NC + lax.axis_index("c")
    base = wid * _ROWS_W
    loc_v = (loc_v0, loc_v1)
    tgt_v = (tgt_v0, tgt_v1)
    msk_v = (msk_v0, msk_v1)
    in_sem = (in_sem0, in_sem1)
    out_sem = (out_sem0, out_sem1)

    def row_off(i):
        return pl.multiple_of(base + i * _CR, _CR)

    def in_copies(i, p):
        r0 = row_off(i)
        return (
            pltpu.make_async_copy(loc_hbm.at[pl.ds(r0, _CR), :], loc_v[p], in_sem[p]),
            pltpu.make_async_copy(tgt_hbm.at[pl.ds(r0, _CR), :], tgt_v[p], in_sem[p]),
            pltpu.make_async_copy(msk_hbm.at[pl.ds(r0, _CR), :], msk_v[p], in_sem[p]),
        )

    def start_in(i, p):
        for d in in_copies(i, p):
            d.start()

    def wait_in(i, p):
        for d in in_copies(i, p):
            d.wait()

    def out_copy(i, p):
        return pltpu.make_async_copy(
            loc_v[p], out_hbm.at[pl.ds(row_off(i), _CR), :], out_sem[p])

    def compute(p):
        lv, tv = loc_v[p], tgt_v[p]
        mw = msk_v[p].bitcast(jnp.int32)

        @plsc.parallel_loop(0, _NGRP, unroll=8)
        def grp_body(g):
            s = g >> 5
            c = (g & 31) * _L
            m_words = mw[s, pl.ds(c, _L)]
            for k in range(4):
                r = s * 4 + k
                sl = pl.ds(c, _L)
                m8 = lax.shift_right_logical(m_words, 8 * k) & 0xFF
                l = lv[r, sl]
                t = tv[r, sl]
                lv[r, sl] = jnp.where(m8 != 0, t, l)

    start_in(0, 0)

    def pair_body(j, carry):
        a = j * 2
        wait_in(a, 0)

        @pl.when(j > 0)
        def _():
            out_copy(a - 2, 0).wait()

        start_in(a + 1, 1)
        compute(0)
        out_copy(a, 0).start()

        wait_in(a + 1, 1)

        @pl.when(j > 0)
        def _():
            out_copy(a - 1, 1).wait()

        @pl.when(j < _NPAIR - 1)
        def _():
            start_in(a + 2, 0)

        compute(1)
        out_copy(a + 1, 1).start()
        return carry

    lax.fori_loop(0, _NPAIR, pair_body, 0)
    out_copy(_NCHUNK - 2, 0).wait()
    out_copy(_NCHUNK - 1, 1).wait()


@jax.jit
def kernel(location, movement_targets, movement_mask):
    mesh = plsc.VectorSubcoreMesh(core_axis_name="c", subcore_axis_name="s")
    out = pl.kernel(
        _body,
        mesh=mesh,
        out_type=jax.ShapeDtypeStruct((_B, _A), jnp.int32),
        scratch_types=[
            pltpu.VMEM((_CR, _A), jnp.int32),
            pltpu.VMEM((_CR, _A), jnp.int32),
            pltpu.VMEM((_CR, _A), jnp.int32),
            pltpu.VMEM((_CR, _A), jnp.int32),
            pltpu.VMEM((_CR, _A), jnp.uint8),
            pltpu.VMEM((_CR, _A), jnp.uint8),
            pltpu.SemaphoreType.DMA,
            pltpu.SemaphoreType.DMA,
            pltpu.SemaphoreType.DMA,
            pltpu.SemaphoreType.DMA,
        ],
        compiler_params=pltpu.CompilerParams(
            use_tc_tiling_on_sc=True, skip_device_barrier=True),
    )(location, movement_targets, movement_mask.view(jnp.uint8))
    return out


# speedup vs baseline: 1.1987x; 1.0024x over previous
import functools

import jax
import jax.numpy as jnp
from jax import lax
from jax.experimental import pallas as pl
from jax.experimental.pallas import tpu as pltpu
from jax.experimental.pallas import tpu_sc as plsc

_B, _A = 16384, 512
_NC, _NS, _L = 2, 16, 16
_NW = _NC * _NS
_ROWS_W = _B // _NW
_CR = 32
_NCHUNK = _ROWS_W // _CR
_NPAIR = _NCHUNK // 2
_NGRP = (_CR // 4) * (_A // _L)


def _body(loc_hbm, tgt_hbm, msk_hbm, out_hbm,
          loc_v0, loc_v1, tgt_v0, tgt_v1, msk_v0, msk_v1,
          in_sem0, in_sem1, out_sem0, out_sem1):
    wid = lax.axis_index("s") * _NC + lax.axis_index("c")
    base = wid * _ROWS_W
    loc_v = (loc_v0, loc_v1)
    tgt_v = (tgt_v0, tgt_v1)
    msk_v = (msk_v0, msk_v1)
    in_sem = (in_sem0, in_sem1)
    out_sem = (out_sem0, out_sem1)

    def row_off(i):
        return pl.multiple_of(base + i * _CR, _CR)

    def in_copies(i, p):
        r0 = row_off(i)
        return (
            pltpu.make_async_copy(loc_hbm.at[pl.ds(r0, _CR), :], loc_v[p], in_sem[p]),
            pltpu.make_async_copy(tgt_hbm.at[pl.ds(r0, _CR), :], tgt_v[p], in_sem[p]),
            pltpu.make_async_copy(msk_hbm.at[pl.ds(r0, _CR), :], msk_v[p], in_sem[p]),
        )

    def start_in(i, p):
        for d in in_copies(i, p):
            d.start()

    def wait_in(i, p):
        for d in in_copies(i, p):
            d.wait()

    def out_copy(i, p):
        return pltpu.make_async_copy(
            loc_v[p], out_hbm.at[pl.ds(row_off(i), _CR), :], out_sem[p])

    lane = lax.iota(jnp.int32, _L)

    def compute(p):
        lv, tv = loc_v[p], tgt_v[p]
        mw = msk_v[p].bitcast(jnp.int32)

        @plsc.parallel_loop(0, _NGRP, unroll=4)
        def grp_body(g):
            s = g >> 5
            c = (g & 31) * _L
            m_words = mw[s, pl.ds(c, _L)]
            cidx = c + lane
            for k in range(4):
                r = s * 4 + k
                sl = pl.ds(c, _L)
                m8 = lax.shift_right_logical(m_words, 8 * k) & 0xFF
                t = tv[r, sl]
                plsc.store_scatter(
                    lv, [jnp.full((_L,), r, jnp.int32), cidx], t, mask=m8 != 0)

    start_in(0, 0)

    def pair_body(j, carry):
        a = j * 2
        wait_in(a, 0)

        @pl.when(j > 0)
        def _():
            out_copy(a - 2, 0).wait()

        start_in(a + 1, 1)
        compute(0)
        out_copy(a, 0).start()

        wait_in(a + 1, 1)

        @pl.when(j > 0)
        def _():
            out_copy(a - 1, 1).wait()

        @pl.when(j < _NPAIR - 1)
        def _():
            start_in(a + 2, 0)

        compute(1)
        out_copy(a + 1, 1).start()
        return carry

    lax.fori_loop(0, _NPAIR, pair_body, 0)
    out_copy(_NCHUNK - 2, 0).wait()
    out_copy(_NCHUNK - 1, 1).wait()


@jax.jit
def kernel(location, movement_targets, movement_mask):
    mesh = plsc.VectorSubcoreMesh(core_axis_name="c", subcore_axis_name="s")
    out = pl.kernel(
        _body,
        mesh=mesh,
        out_type=jax.ShapeDtypeStruct((_B, _A), jnp.int32),
        scratch_types=[
            pltpu.VMEM((_CR, _A), jnp.int32),
            pltpu.VMEM((_CR, _A), jnp.int32),
            pltpu.VMEM((_CR, _A), jnp.int32),
            pltpu.VMEM((_CR, _A), jnp.int32),
            pltpu.VMEM((_CR, _A), jnp.uint8),
            pltpu.VMEM((_CR, _A), jnp.uint8),
            pltpu.SemaphoreType.DMA,
            pltpu.SemaphoreType.DMA,
            pltpu.SemaphoreType.DMA,
            pltpu.SemaphoreType.DMA,
        ],
        compiler_params=pltpu.CompilerParams(
            use_tc_tiling_on_sc=True, needs_layout_passes=False),
    )(location, movement_targets, movement_mask.view(jnp.uint8))
    return out


# speedup vs baseline: 1.2141x; 1.0128x over previous
import functools

import jax
import jax.numpy as jnp
from jax import lax
from jax.experimental import pallas as pl
from jax.experimental.pallas import tpu as pltpu
from jax.experimental.pallas import tpu_sc as plsc

_B, _A = 16384, 512
_NC, _NS, _L = 2, 16, 16
_NW = _NC * _NS
_ROWS_W = _B // _NW
_CR = 32
_NCHUNK = _ROWS_W // _CR
_NPAIR = _NCHUNK // 2
_NGRP = (_CR // 4) * (_A // _L)


def _body(loc_hbm, tgt_hbm, msk_hbm, out_hbm,
          loc_v0, loc_v1, tgt_v0, tgt_v1, msk_v0, msk_v1,
          in_sem0, in_sem1, out_sem0, out_sem1):
    wid = lax.axis_index("s") * _NC + lax.axis_index("c")
    base = wid * _ROWS_W
    loc_v = (loc_v0, loc_v1)
    tgt_v = (tgt_v0, tgt_v1)
    msk_v = (msk_v0, msk_v1)
    in_sem = (in_sem0, in_sem1)
    out_sem = (out_sem0, out_sem1)

    def row_off(i):
        return pl.multiple_of(base + i * _CR, _CR)

    def in_copies(i, p):
        r0 = row_off(i)
        return (
            pltpu.make_async_copy(loc_hbm.at[pl.ds(r0, _CR), :], loc_v[p], in_sem[p]),
            pltpu.make_async_copy(tgt_hbm.at[pl.ds(r0, _CR), :], tgt_v[p], in_sem[p]),
            pltpu.make_async_copy(msk_hbm.at[pl.ds(r0, _CR), :], msk_v[p], in_sem[p]),
        )

    def start_in(i, p):
        for d in in_copies(i, p):
            d.start()

    def wait_in(i, p):
        for d in in_copies(i, p):
            d.wait()

    def out_copy(i, p):
        return pltpu.make_async_copy(
            loc_v[p], out_hbm.at[pl.ds(row_off(i), _CR), :], out_sem[p])

    def compute(p):
        lv, tv = loc_v[p], tgt_v[p]
        mw = msk_v[p].bitcast(jnp.int32)

        @plsc.parallel_loop(0, _NGRP, unroll=4)
        def grp_body(g):
            s = g >> 5
            c = (g & 31) * _L
            m_words = mw[s, pl.ds(c, _L)]
            for k in range(4):
                r = s * 4 + k
                sl = pl.ds(c, _L)
                m8 = lax.shift_right_logical(m_words, 8 * k) & 0xFF
                l = lv[r, sl]
                t = tv[r, sl]
                lv[r, sl] = jnp.where(m8 != 0, t, l)

    start_in(0, 0)

    def pair_body(j, carry):
        a = j * 2
        wait_in(a, 0)

        @pl.when(j > 0)
        def _():
            out_copy(a - 2, 0).wait()

        start_in(a + 1, 1)
        compute(0)
        out_copy(a, 0).start()

        wait_in(a + 1, 1)

        @pl.when(j > 0)
        def _():
            out_copy(a - 1, 1).wait()

        @pl.when(j < _NPAIR - 1)
        def _():
            start_in(a + 2, 0)

        compute(1)
        out_copy(a + 1, 1).start()
        return carry

    lax.fori_loop(0, _NPAIR, pair_body, 0)
    out_copy(_NCHUNK - 2, 0).wait()
    out_copy(_NCHUNK - 1, 1).wait()


@jax.jit
def kernel(location, movement_targets, movement_mask):
    mesh = plsc.VectorSubcoreMesh(core_axis_name="c", subcore_axis_name="s")
    out = pl.kernel(
        _body,
        mesh=mesh,
        out_type=jax.ShapeDtypeStruct((_B, _A), jnp.int32),
        scratch_types=[
            pltpu.VMEM((_CR, _A), jnp.int32),
            pltpu.VMEM((_CR, _A), jnp.int32),
            pltpu.VMEM((_CR, _A), jnp.int32),
            pltpu.VMEM((_CR, _A), jnp.int32),
            pltpu.VMEM((_CR, _A), jnp.uint8),
            pltpu.VMEM((_CR, _A), jnp.uint8),
            pltpu.SemaphoreType.DMA,
            pltpu.SemaphoreType.DMA,
            pltpu.SemaphoreType.DMA,
            pltpu.SemaphoreType.DMA,
        ],
        compiler_params=pltpu.CompilerParams(use_tc_tiling_on_sc=True),
    )(location, movement_targets, movement_mask.view(jnp.uint8))
    return out


# speedup vs baseline: 1.2143x; 1.0001x over previous
import functools

import jax
import jax.numpy as jnp
from jax import lax
from jax.experimental import pallas as pl
from jax.experimental.pallas import tpu as pltpu
from jax.experimental.pallas import tpu_sc as plsc

_B, _A = 16384, 512
_NC, _NS, _L = 2, 16, 16
_NW = _NC * _NS
_ROWS_W = _B // _NW
_CR = 32
_NCHUNK = _ROWS_W // _CR
_NPAIR = _NCHUNK // 2
_NGRP = (_CR // 4) * (_A // _L)


def _body(loc_hbm, tgt_hbm, msk_hbm, out_hbm,
          loc_v0, loc_v1, tgt_v0, tgt_v1, msk_v0, msk_v1,
          in_sem0, in_sem1, out_sem0, out_sem1):
    wid = lax.axis_index("s") * _NC + lax.axis_index("c")
    base = wid * _ROWS_W
    loc_v = (loc_v0, loc_v1)
    tgt_v = (tgt_v0, tgt_v1)
    msk_v = (msk_v0, msk_v1)
    in_sem = (in_sem0, in_sem1)
    out_sem = (out_sem0, out_sem1)

    def row_off(i):
        return pl.multiple_of(base + i * _CR, _CR)

    def in_copies(i, p):
        r0 = row_off(i)
        return (
            pltpu.make_async_copy(loc_hbm.at[pl.ds(r0, _CR), :], loc_v[p], in_sem[p]),
            pltpu.make_async_copy(tgt_hbm.at[pl.ds(r0, _CR), :], tgt_v[p], in_sem[p]),
            pltpu.make_async_copy(msk_hbm.at[pl.ds(r0, _CR), :], msk_v[p], in_sem[p]),
        )

    def start_in(i, p):
        for d in in_copies(i, p):
            d.start()

    def wait_in(i, p):
        for d in in_copies(i, p):
            d.wait()

    def out_copy(i, p):
        return pltpu.make_async_copy(
            loc_v[p], out_hbm.at[pl.ds(row_off(i), _CR), :], out_sem[p])

    def compute(p):
        lv, tv = loc_v[p], tgt_v[p]
        mw = msk_v[p].bitcast(jnp.int32)

        @plsc.parallel_loop(0, _NGRP, unroll=4)
        def grp_body(g):
            s = g >> 5
            c = (g & 31) * _L
            m_words = mw[s, pl.ds(c, _L)]
            for k in range(4):
                r = s * 4 + k
                sl = pl.ds(c, _L)
                m8 = lax.shift_right_logical(m_words, 8 * k) & 0xFF
                l = lv[r, sl]
                t = tv[r, sl]
                lv[r, sl] = jnp.where(m8 != 0, t, l)

    start_in(0, 0)

    def pair_body(j, carry):
        a = j * 2
        wait_in(a, 0)

        @pl.when(j > 0)
        def _():
            out_copy(a - 2, 0).wait()

        start_in(a + 1, 1)
        compute(0)
        out_copy(a, 0).start()

        wait_in(a + 1, 1)

        @pl.when(j > 0)
        def _():
            out_copy(a - 1, 1).wait()

        @pl.when(j < _NPAIR - 1)
        def _():
            start_in(a + 2, 0)

        compute(1)
        out_copy(a + 1, 1).start()
        return carry

    lax.fori_loop(0, _NPAIR, pair_body, 0)
    out_copy(_NCHUNK - 2, 0).wait()
    out_copy(_NCHUNK - 1, 1).wait()


@jax.jit
def kernel(location, movement_targets, movement_mask):
    mesh = plsc.VectorSubcoreMesh(core_axis_name="c", subcore_axis_name="s")
    out = pl.kernel(
        _body,
        mesh=mesh,
        out_type=jax.ShapeDtypeStruct((_B, _A), jnp.int32),
        scratch_types=[
            pltpu.VMEM((_CR, _A), jnp.int32),
            pltpu.VMEM((_CR, _A), jnp.int32),
            pltpu.VMEM((_CR, _A), jnp.int32),
            pltpu.VMEM((_CR, _A), jnp.int32),
            pltpu.VMEM((_CR, _A), jnp.uint8),
            pltpu.VMEM((_CR, _A), jnp.uint8),
            pltpu.SemaphoreType.DMA,
            pltpu.SemaphoreType.DMA,
            pltpu.SemaphoreType.DMA,
            pltpu.SemaphoreType.DMA,
        ],
        compiler_params=pltpu.CompilerParams(
            use_tc_tiling_on_sc=True, skip_device_barrier=True),
    )(location, movement_targets, movement_mask.view(jnp.uint8))
    return out
